# baseline (speedup 1.0000x reference)
import jax
import jax.numpy as jnp
from jax.experimental import pallas as pl
from jax.experimental.pallas import tpu as pltpu

CDT = jnp.bfloat16
PAR = ((0, 0), (0, 1), (1, 0), (1, 1))
B = 8

L1_IN = 102 * 128
C1_OUT = 100 * 128
P2_L = 3328
C2_OUT = 50 * 64
P3_L = 896
C3_OUT = 25 * 32


def _chunks(total, ch):
    t, out = 0, []
    while t < total:
        out.append((t, min(ch, total - t)))
        t += ch
    return out


def _conv_body(x_ref, l_ref, b_ref, o_ref, *, offs, stride, width,
               out_lanes, chunk):
    nb = x_ref.shape[0]

    def body(i, _):
        for t0, ch in _chunks(out_lanes, chunk):
            rhs = jnp.concatenate(
                [x_ref[i, :, o + t0:o + t0 + ch] for o in offs], axis=0)
            acc = jnp.dot(l_ref[...], rhs,
                          preferred_element_type=jnp.float32)
            pooled = jnp.maximum(jnp.maximum(acc[0:16], acc[16:32]),
                                 jnp.maximum(acc[32:48], acc[48:64]))
            pooled = jnp.maximum(pooled + b_ref[...], 0.0)
            lane = jax.lax.broadcasted_iota(jnp.int32, (16, ch), 1) + t0
            pooled = jnp.where((lane & (stride - 1)) < width, pooled, 0.0)
            o_ref[i, :, t0:t0 + ch] = pooled.astype(CDT)
        return 0

    jax.lax.fori_loop(0, nb, body, 0)


def _conv3_fc_body(x_ref, l3_ref, b3_ref, w1_ref, c1_ref, w2_ref, c2_ref,
                   w3_ref, c3_ref, w4_ref, c4_ref, o_ref, fcin_ref):
    nb = x_ref.shape[0]

    def body(i, _):
        rhs = jnp.concatenate(
            [x_ref[i, :, o:o + C3_OUT] for o in (0, 1, 32, 33)], axis=0)
        acc = jnp.dot(l3_ref[...], rhs, preferred_element_type=jnp.float32)
        pooled = jnp.maximum(jnp.maximum(acc[0:16], acc[16:32]),
                             jnp.maximum(acc[32:48], acc[48:64]))
        pooled = jnp.maximum(pooled + b3_ref[...], 0.0)
        lane = jax.lax.broadcasted_iota(jnp.int32, (16, C3_OUT), 1)
        pooled = jnp.where((lane & 31) < 25, pooled, 0.0)
        fcin_ref[i] = pooled.astype(CDT)
        return 0

    jax.lax.fori_loop(0, nb, body, 0)

    h = jnp.zeros((nb, 120), jnp.float32)
    for c in range(10):
        h = h + jnp.dot(fcin_ref[:, c, :], w1_ref[c],
                        preferred_element_type=jnp.float32)
    h = jnp.maximum(h + c1_ref[...], 0.0)
    h = jnp.dot(h.astype(CDT), w2_ref[...],
                preferred_element_type=jnp.float32) + c2_ref[...]
    h = jnp.maximum(h, 0.0)
    h = jnp.dot(h.astype(CDT), w3_ref[...],
                preferred_element_type=jnp.float32) + c3_ref[...]
    h = jnp.maximum(h, 0.0)
    o_ref[...] = jnp.dot(h.astype(CDT), w4_ref[...],
                         preferred_element_type=jnp.float32) + c4_ref[...]


def _pack_conv_lhs(lmat, cin):
    cpad = 4 if cin == 3 else 16
    w = lmat.reshape(4, 4, 10, 4, cin)
    w = jnp.pad(w, ((0, 0), (0, 0), (0, 6), (0, 0), (0, cpad - cin)))
    w = w.transpose(1, 2, 0, 3, 4)
    return w.reshape(64, 4 * 4 * cpad).astype(CDT)


def _pad_bias(b):
    return jnp.pad(b.astype(jnp.float32).reshape(-1, 1), ((0, 6), (0, 0)))


def _parity_split(y, hh, srow, scol):
    N = y.shape[0]
    h2 = hh // 2 + 1
    yp = jnp.pad(y, ((0, 0), (0, 0), (1, 1), (1, 1)))
    parts = jnp.stack([yp[:, :, pu::2, pv::2] for pu, pv in PAR], axis=1)
    parts = jnp.pad(parts, ((0, 0), (0, 0), (0, 0), (0, 0), (0, scol - h2)))
    return parts.reshape(N, 64, h2 * scol)


def _const_spec(s):
    return pl.BlockSpec(s, lambda n: tuple(0 for _ in s))


def _conv_call(x, l, b, offs, stride, width, out_lanes, chunk, in_lanes):
    N = x.shape[0]
    return pl.pallas_call(
        lambda x_ref, l_ref, b_ref, o_ref: _conv_body(
            x_ref, l_ref, b_ref, o_ref, offs=offs, stride=stride,
            width=width, out_lanes=out_lanes, chunk=chunk),
        out_shape=jax.ShapeDtypeStruct((N, 16, out_lanes), CDT),
        grid_spec=pltpu.PrefetchScalarGridSpec(
            num_scalar_prefetch=0,
            grid=(N // B,),
            in_specs=[
                pl.BlockSpec((B, 64 if l.shape[1] == 256 else 16, in_lanes),
                             lambda n: (n, 0, 0)),
                _const_spec(l.shape), _const_spec((16, 1)),
            ],
            out_specs=pl.BlockSpec((B, 16, out_lanes), lambda n: (n, 0, 0)),
        ),
        compiler_params=pltpu.CompilerParams(
            dimension_semantics=("parallel",),
            vmem_limit_bytes=60 * 1024 * 1024),
    )(x, l, b)


def kernel(x, conv1_l, conv1_b, conv2_l, conv2_b, conv3_l, conv3_b,
           fc1_w, fc1_b, fc2_w, fc2_b, fc3_w, fc3_b, fc4_w, fc4_b):
    N = x.shape[0]

    xp = jnp.pad(x.astype(CDT), ((0, 0), (0, 0), (1, 1), (1, 1)))
    parts = jnp.stack([xp[:, :, pu::2, pv::2] for pu, pv in PAR], axis=1)
    parts = jnp.pad(parts, ((0, 0), (0, 0), (0, 1), (0, 1), (0, 27)))
    xs = parts.reshape(N, 16, L1_IN)

    l1 = _pack_conv_lhs(conv1_l, 3)
    l2 = _pack_conv_lhs(conv2_l, 10)
    l3 = _pack_conv_lhs(conv3_l, 10)
    b1, b2, b3 = _pad_bias(conv1_b), _pad_bias(conv2_b), _pad_bias(conv3_b)

    w1 = fc1_w.astype(CDT).reshape(10, 25, 25, 120)
    w1 = jnp.pad(w1, ((0, 0), (0, 0), (0, 7), (0, 0))).reshape(10, C3_OUT, 120)
    w2, w3, w4 = (w.astype(CDT) for w in (fc2_w, fc3_w, fc4_w))
    c1, c2, c3, c4 = (v.astype(jnp.float32).reshape(1, -1)
                      for v in (fc1_b, fc2_b, fc3_b, fc4_b))

    y1 = _conv_call(xs, l1, b1, (0, 1, 128, 129), 128, 100, C1_OUT, 1280,
                    L1_IN)
    p2 = _parity_split(y1.reshape(N, 16, 100, 128)[:, :, :, :100],
                       100, 64, 64)
    p2 = jnp.pad(p2, ((0, 0), (0, 0), (0, P2_L - p2.shape[2])))

    y2 = _conv_call(p2, l2, b2, (0, 1, 64, 65), 64, 50, C2_OUT, 1280, P2_L)
    p3 = _parity_split(y2.reshape(N, 16, 50, 64)[:, :, :, :50], 50, 32, 32)
    p3 = jnp.pad(p3, ((0, 0), (0, 0), (0, P3_L - p3.shape[2])))

    return pl.pallas_call(
        _conv3_fc_body,
        out_shape=jax.ShapeDtypeStruct((N, 5), jnp.float32),
        grid_spec=pltpu.PrefetchScalarGridSpec(
            num_scalar_prefetch=0,
            grid=(N // B,),
            in_specs=[
                pl.BlockSpec((B, 64, P3_L), lambda n: (n, 0, 0)),
                _const_spec((64, 256)), _const_spec((16, 1)),
                _const_spec((10, C3_OUT, 120)), _const_spec((1, 120)),
                _const_spec((120, 80)), _const_spec((1, 80)),
                _const_spec((80, 40)), _const_spec((1, 40)),
                _const_spec((40, 5)), _const_spec((1, 5)),
            ],
            out_specs=pl.BlockSpec((B, 5), lambda n: (n, 0)),
            scratch_shapes=[pltpu.VMEM((B, 16, C3_OUT), CDT)],
        ),
        compiler_params=pltpu.CompilerParams(
            dimension_semantics=("parallel",),
            vmem_limit_bytes=60 * 1024 * 1024),
    )(p3, l3, b3, w1, c1, w2, c2, w3, c3, w4, c4)


# speedup vs baseline: 3.7670x; 3.7670x over previous
import jax
import jax.numpy as jnp
from jax.experimental import pallas as pl
from jax.experimental.pallas import tpu as pltpu

CDT = jnp.bfloat16
PAR = ((0, 0), (0, 1), (1, 0), (1, 1))
B = 8

L1_IN = 102 * 128
C1_OUT = 100 * 128
P2_L = 3328
C2_OUT = 50 * 64
P3_L = 896
C3_OUT = 25 * 32


def _chunks(total, ch):
    t, out = 0, []
    while t < total:
        out.append((t, min(ch, total - t)))
        t += ch
    return out


def _conv_body(x_ref, l_ref, b_ref, o_ref, *, offs, stride, width,
               out_lanes, chunk):
    nb = x_ref.shape[0]

    def body(i, _):
        for t0, ch in _chunks(out_lanes, chunk):
            rhs = jnp.concatenate(
                [x_ref[i, :, o + t0:o + t0 + ch] for o in offs], axis=0)
            acc = jnp.dot(l_ref[...], rhs,
                          preferred_element_type=jnp.float32)
            pooled = jnp.maximum(jnp.maximum(acc[0:16], acc[16:32]),
                                 jnp.maximum(acc[32:48], acc[48:64]))
            pooled = jnp.maximum(pooled + b_ref[...], 0.0)
            lane = jax.lax.broadcasted_iota(jnp.int32, (16, ch), 1) + t0
            pooled = jnp.where((lane & (stride - 1)) < width, pooled, 0.0)
            o_ref[i, :, t0:t0 + ch] = pooled.astype(CDT)
        return 0

    jax.lax.fori_loop(0, nb, body, 0)


def _conv3_fc_body(x_ref, l3_ref, b3_ref, w1_ref, c1_ref, w2_ref, c2_ref,
                   w3_ref, c3_ref, w4_ref, c4_ref, o_ref, fcin_ref):
    nb = x_ref.shape[0]

    def body(i, _):
        rhs = jnp.concatenate(
            [x_ref[i, :, o:o + C3_OUT] for o in (0, 1, 32, 33)], axis=0)
        acc = jnp.dot(l3_ref[...], rhs, preferred_element_type=jnp.float32)
        pooled = jnp.maximum(jnp.maximum(acc[0:16], acc[16:32]),
                             jnp.maximum(acc[32:48], acc[48:64]))
        pooled = jnp.maximum(pooled + b3_ref[...], 0.0)
        lane = jax.lax.broadcasted_iota(jnp.int32, (16, C3_OUT), 1)
        pooled = jnp.where((lane & 31) < 25, pooled, 0.0)
        fcin_ref[i] = pooled.astype(CDT)
        return 0

    jax.lax.fori_loop(0, nb, body, 0)

    h = jnp.zeros((nb, 120), jnp.float32)
    for c in range(10):
        h = h + jnp.dot(fcin_ref[:, c, :], w1_ref[c],
                        preferred_element_type=jnp.float32)
    h = jnp.maximum(h + c1_ref[...], 0.0)
    h = jnp.dot(h.astype(CDT), w2_ref[...],
                preferred_element_type=jnp.float32) + c2_ref[...]
    h = jnp.maximum(h, 0.0)
    h = jnp.dot(h.astype(CDT), w3_ref[...],
                preferred_element_type=jnp.float32) + c3_ref[...]
    h = jnp.maximum(h, 0.0)
    o_ref[...] = jnp.dot(h.astype(CDT), w4_ref[...],
                         preferred_element_type=jnp.float32) + c4_ref[...]


def _pack_conv_lhs(lmat, cin):
    cpad = 4 if cin == 3 else 16
    w = lmat.reshape(4, 4, 10, 4, cin)
    w = jnp.pad(w, ((0, 0), (0, 0), (0, 6), (0, 0), (0, cpad - cin)))
    w = w.transpose(1, 2, 0, 3, 4)
    return w.reshape(64, 4 * 4 * cpad).astype(CDT)


def _pad_bias(b):
    return jnp.pad(b.astype(jnp.float32).reshape(-1, 1), ((0, 6), (0, 0)))


def _parity_split(y, hh, srow, scol):
    N = y.shape[0]
    h2 = hh // 2 + 1
    yp = jnp.pad(y, ((0, 0), (0, 0), (1, 1), (1, 1)))
    parts = yp.reshape(N, 16, h2, 2, h2, 2).transpose(0, 3, 5, 1, 2, 4)
    parts = jnp.pad(parts,
                    ((0, 0), (0, 0), (0, 0), (0, 0), (0, 0), (0, scol - h2)))
    return parts.reshape(N, 64, h2 * scol)


def _const_spec(s):
    return pl.BlockSpec(s, lambda n: tuple(0 for _ in s))


def _conv_call(x, l, b, offs, stride, width, out_lanes, chunk, in_lanes):
    N = x.shape[0]
    return pl.pallas_call(
        lambda x_ref, l_ref, b_ref, o_ref: _conv_body(
            x_ref, l_ref, b_ref, o_ref, offs=offs, stride=stride,
            width=width, out_lanes=out_lanes, chunk=chunk),
        out_shape=jax.ShapeDtypeStruct((N, 16, out_lanes), CDT),
        grid_spec=pltpu.PrefetchScalarGridSpec(
            num_scalar_prefetch=0,
            grid=(N // B,),
            in_specs=[
                pl.BlockSpec((B, 64 if l.shape[1] == 256 else 16, in_lanes),
                             lambda n: (n, 0, 0)),
                _const_spec(l.shape), _const_spec((16, 1)),
            ],
            out_specs=pl.BlockSpec((B, 16, out_lanes), lambda n: (n, 0, 0)),
        ),
        compiler_params=pltpu.CompilerParams(
            dimension_semantics=("parallel",),
            vmem_limit_bytes=60 * 1024 * 1024),
    )(x, l, b)


def kernel(x, conv1_l, conv1_b, conv2_l, conv2_b, conv3_l, conv3_b,
           fc1_w, fc1_b, fc2_w, fc2_b, fc3_w, fc3_b, fc4_w, fc4_b):
    N = x.shape[0]

    xp = jnp.pad(x.astype(CDT), ((0, 0), (0, 0), (1, 1), (1, 1)))
    parts = xp.reshape(N, 3, 101, 2, 101, 2).transpose(0, 3, 5, 1, 2, 4)
    parts = jnp.pad(parts,
                    ((0, 0), (0, 0), (0, 0), (0, 1), (0, 1), (0, 27)))
    xs = parts.reshape(N, 16, L1_IN)

    l1 = _pack_conv_lhs(conv1_l, 3)
    l2 = _pack_conv_lhs(conv2_l, 10)
    l3 = _pack_conv_lhs(conv3_l, 10)
    b1, b2, b3 = _pad_bias(conv1_b), _pad_bias(conv2_b), _pad_bias(conv3_b)

    w1 = fc1_w.astype(CDT).reshape(10, 25, 25, 120)
    w1 = jnp.pad(w1, ((0, 0), (0, 0), (0, 7), (0, 0))).reshape(10, C3_OUT, 120)
    w2, w3, w4 = (w.astype(CDT) for w in (fc2_w, fc3_w, fc4_w))
    c1, c2, c3, c4 = (v.astype(jnp.float32).reshape(1, -1)
                      for v in (fc1_b, fc2_b, fc3_b, fc4_b))

    y1 = _conv_call(xs, l1, b1, (0, 1, 128, 129), 128, 100, C1_OUT, 1280,
                    L1_IN)
    p2 = _parity_split(y1.reshape(N, 16, 100, 128)[:, :, :, :100],
                       100, 64, 64)
    p2 = jnp.pad(p2, ((0, 0), (0, 0), (0, P2_L - p2.shape[2])))

    y2 = _conv_call(p2, l2, b2, (0, 1, 64, 65), 64, 50, C2_OUT, 1280, P2_L)
    p3 = _parity_split(y2.reshape(N, 16, 50, 64)[:, :, :, :50], 50, 32, 32)
    p3 = jnp.pad(p3, ((0, 0), (0, 0), (0, P3_L - p3.shape[2])))

    return pl.pallas_call(
        _conv3_fc_body,
        out_shape=jax.ShapeDtypeStruct((N, 5), jnp.float32),
        grid_spec=pltpu.PrefetchScalarGridSpec(
            num_scalar_prefetch=0,
            grid=(N // B,),
            in_specs=[
                pl.BlockSpec((B, 64, P3_L), lambda n: (n, 0, 0)),
                _const_spec((64, 256)), _const_spec((16, 1)),
                _const_spec((10, C3_OUT, 120)), _const_spec((1, 120)),
                _const_spec((120, 80)), _const_spec((1, 80)),
                _const_spec((80, 40)), _const_spec((1, 40)),
                _const_spec((40, 5)), _const_spec((1, 5)),
            ],
            out_specs=pl.BlockSpec((B, 5), lambda n: (n, 0)),
            scratch_shapes=[pltpu.VMEM((B, 16, C3_OUT), CDT)],
        ),
        compiler_params=pltpu.CompilerParams(
            dimension_semantics=("parallel",),
            vmem_limit_bytes=60 * 1024 * 1024),
    )(p3, l3, b3, w1, c1, w2, c2, w3, c3, w4, c4)
